# Initial kernel scaffold; baseline (speedup 1.0000x reference)
import jax
import jax.numpy as jnp
from jax.experimental import pallas as pl


def kernel(x, edge_index, eps, W1, b1, g1, be1, W2, b2, g2, be2, bn_g, bn_b):
    raise NotImplementedError("write your pallas kernel here")


# speedup vs baseline: 3.5034x; 3.5034x over previous
import functools

import jax
import jax.numpy as jnp
from jax import lax
from jax.experimental import pallas as pl
from jax.experimental.pallas import tpu as pltpu
from jax.experimental.pallas import tpu_sc as plsc

_NC = 2
_NS = 16
_NW = _NC * _NS
_CHUNK = 128


def _make_segment_sum(n, d, e_pad, agg_rows):
    per_w = e_pad // _NW
    n_chunks = per_w // _CHUNK
    rows_per_tile = agg_rows // _NS
    mesh = plsc.VectorSubcoreMesh(core_axis_name="c", subcore_axis_name="s")

    def body(h_hbm, src_hbm, dst_hbm, zero_hbm, out_hbm,
             agg_sh, src_v, dst_v, rows_v, sem):
        c = lax.axis_index("c")
        s = lax.axis_index("s")
        wid = s * _NC + c
        pltpu.sync_copy(zero_hbm, agg_sh.at[pl.ds(s * rows_per_tile, rows_per_tile)])
        plsc.subcore_barrier()

        base = wid * per_w

        def chunk(k, carry):
            off = base + k * _CHUNK
            pltpu.sync_copy(src_hbm.at[pl.ds(off, _CHUNK)], src_v)
            pltpu.sync_copy(dst_hbm.at[pl.ds(off, _CHUNK)], dst_v)
            pltpu.async_copy(h_hbm.at[src_v], rows_v, sem).wait()
            pltpu.sync_copy(rows_v, agg_sh.at[dst_v], add=True)
            return carry

        lax.fori_loop(0, n_chunks, chunk, 0)
        plsc.subcore_barrier()
        pltpu.sync_copy(agg_sh.at[pl.ds(s * rows_per_tile, rows_per_tile)],
                        out_hbm.at[c, pl.ds(s * rows_per_tile, rows_per_tile)])

    return pl.kernel(
        body,
        out_type=jax.ShapeDtypeStruct((_NC, agg_rows, d), jnp.float32),
        mesh=mesh,
        scratch_types=[
            pltpu.VMEM_SHARED((agg_rows, d), jnp.float32),
            pltpu.VMEM((_CHUNK,), jnp.int32),
            pltpu.VMEM((_CHUNK,), jnp.int32),
            pltpu.VMEM((_CHUNK, d), jnp.float32),
            pltpu.SemaphoreType.DMA,
        ],
    )


def _bn(x, g, b):
    mu = jnp.mean(x, axis=0, keepdims=True)
    var = jnp.mean((x - mu) ** 2, axis=0, keepdims=True)
    return g * (x - mu) * lax.rsqrt(var + 1e-5) + b


def _dense_body(scale_ref, h_ref, p0_ref, p1_ref, w1_ref, b1_ref, g1_ref,
                be1_ref, w2_ref, b2_ref, g2_ref, be2_ref, bng_ref, bnb_ref,
                out_ref, *, n, final_relu):
    agg = p0_ref[0:n, :] + p1_ref[0:n, :]
    h2 = h_ref[...] * scale_ref[0] + agg
    a = jnp.dot(h2, w1_ref[...], preferred_element_type=jnp.float32) + b1_ref[...]
    a = jnp.maximum(_bn(a, g1_ref[...], be1_ref[...]), 0.0)
    a = jnp.dot(a, w2_ref[...], preferred_element_type=jnp.float32) + b2_ref[...]
    a = jnp.maximum(_bn(a, g2_ref[...], be2_ref[...]), 0.0)
    a = _bn(a, bng_ref[...], bnb_ref[...])
    if final_relu:
        a = jnp.maximum(a, 0.0)
    out_ref[...] = a


def _make_dense(n, d, final_relu):
    vmem = pl.BlockSpec(memory_space=pltpu.VMEM)
    return pl.pallas_call(
        functools.partial(_dense_body, n=n, final_relu=final_relu),
        out_shape=jax.ShapeDtypeStruct((n, d), jnp.float32),
        in_specs=[pl.BlockSpec(memory_space=pltpu.SMEM)] + [vmem] * 13,
        out_specs=vmem,
    )


def kernel(x, edge_index, eps, W1, b1, g1, be1, W2, b2, g2, be2, bn_g, bn_b):
    n, d = x.shape
    e = edge_index.shape[1]
    num_layers = W1.shape[0]

    agg_rows = ((n + _NS - 1) // _NS + 7) // 8 * 8 * _NS
    junk_row = n
    e_pad = ((e + _NW * _CHUNK - 1) // (_NW * _CHUNK)) * (_NW * _CHUNK)

    src = edge_index[0]
    dst = edge_index[1]
    pad = e_pad - e
    src_p = jnp.concatenate([src, jnp.zeros((pad,), jnp.int32)])
    dst_p = jnp.concatenate([dst, jnp.full((pad,), junk_row, jnp.int32)])
    zero_block = jnp.zeros((agg_rows // _NS, d), jnp.float32)

    seg_sum = _make_segment_sum(n, d, e_pad, agg_rows)

    h = x
    for i in range(num_layers):
        parts = seg_sum(h, src_p, dst_p, zero_block)
        scale = (1.0 + eps[i]).reshape(1)
        dense = _make_dense(n, d, final_relu=(i < num_layers - 1))
        h = dense(scale, h, parts[0], parts[1],
                  W1[i], b1[i].reshape(1, d), g1[i].reshape(1, d),
                  be1[i].reshape(1, d), W2[i], b2[i].reshape(1, d),
                  g2[i].reshape(1, d), be2[i].reshape(1, d),
                  bn_g[i].reshape(1, d), bn_b[i].reshape(1, d))
    return h
